# Initial kernel scaffold; baseline (speedup 1.0000x reference)
import jax
import jax.numpy as jnp
from jax.experimental import pallas as pl


def kernel(h, idx2, idx3, idx4, W1_2, b1_2, Wo_2, bo_2, W1_3, b1_3, Wo_3, bo_3, W1_4, b1_4, Wo_4, bo_4):
    raise NotImplementedError("write your pallas kernel here")


# speedup vs baseline: 3.3078x; 3.3078x over previous
import functools

import jax
import jax.numpy as jnp
from jax import lax
from jax.experimental import pallas as pl
from jax.experimental.pallas import tpu as pltpu
from jax.experimental.pallas import tpu_sc as plsc

N1 = 50000
D = 128
HID = 64
N2, N3, N4 = 40000, 60000, 80000
NC, NS = 2, 16
NW = NC * NS
CH = 128
N2P, N3P, N4P = 40960, 61440, 81920

_f32 = jnp.float32


def _tables_body(h_ref, wc_ref, *out_refs):
    x = h_ref[...]
    for t, o_ref in enumerate(out_refs):
        o_ref[...] = jnp.dot(x, wc_ref[:, t * HID:(t + 1) * HID],
                             preferred_element_type=_f32)


def _make_tables(h, wc):
    blk = 1000
    grid = (N1 // blk,)
    return pl.pallas_call(
        _tables_body,
        grid=grid,
        in_specs=[
            pl.BlockSpec((blk, D), lambda i: (i, 0)),
            pl.BlockSpec((D, 5 * HID), lambda i: (0, 0)),
        ],
        out_specs=[pl.BlockSpec((blk, HID), lambda i: (i, 0))] * 5,
        out_shape=[jax.ShapeDtypeStruct((N1, HID), _f32)] * 5,
    )(h, wc)


_LEVELS = ((N2P, (0, 0)), (N3P, (1, 2, 1)), (N4P, (3, 4, 4, 3)))


def _sc_body(t2, t3a, t3b, t4a, t4b, g2, g3, g4, s2, s3, s4,
             ibuf, rb0, rb1, rb2, rb3, sem):
    tables = (t2, t3a, t3b, t4a, t4b)
    rbufs = (rb0, rb1, rb2, rb3)
    wid = lax.axis_index("s") * NC + lax.axis_index("c")
    for (npad, slots), gidx, s_out in zip(_LEVELS, (g2, g3, g4), (s2, s3, s4)):
        L = len(slots)
        nch = npad // (NW * CH)

        def chunk_body(c, _, L=L, slots=slots, nch=nch, gidx=gidx, s_out=s_out):
            g = wid * nch + c
            pltpu.sync_copy(gidx.at[g], ibuf.at[pl.ds(0, L)])
            cps = [pltpu.async_copy(tables[slots[r]].at[ibuf.at[r]],
                                    rbufs[r], sem) for r in range(L)]
            for cp in cps:
                cp.wait()

            def acc_body(j, _, L=L):
                for seg in range(HID // 16):
                    sl = pl.ds(seg * 16, 16)
                    for r in range(1, L):
                        plsc.addupdate(rb0.at[j, sl], rbufs[r][j, sl])
                return 0

            lax.fori_loop(0, CH, acc_body, 0)
            pltpu.sync_copy(rb0, s_out.at[pl.ds(g * CH, CH)])
            return 0

        lax.fori_loop(0, nch, chunk_body, 0)


def _sc_gather_sum(tables, g2, g3, g4):
    mesh = plsc.VectorSubcoreMesh(core_axis_name="c", subcore_axis_name="s",
                                  num_cores=NC, num_subcores=NS)
    fn = pl.kernel(
        _sc_body,
        out_type=[jax.ShapeDtypeStruct((N2P, HID), _f32),
                  jax.ShapeDtypeStruct((N3P, HID), _f32),
                  jax.ShapeDtypeStruct((N4P, HID), _f32)],
        mesh=mesh,
        scratch_types=[
            pltpu.VMEM((4, CH), jnp.int32),
            pltpu.VMEM((CH, HID), _f32),
            pltpu.VMEM((CH, HID), _f32),
            pltpu.VMEM((CH, HID), _f32),
            pltpu.VMEM((CH, HID), _f32),
            pltpu.SemaphoreType.DMA,
        ],
        compiler_params=pltpu.CompilerParams(use_tc_tiling_on_sc=False),
    )
    return fn(*tables, g2, g3, g4)


def _head_body(s_ref, b1_ref, wo_ref, bo_ref, o_ref):
    y = jnp.maximum(s_ref[...] + b1_ref[...], 0.0)
    o_ref[...] = jnp.dot(y, wo_ref[...], preferred_element_type=_f32) \
        + bo_ref[...]


def _head(s, b1, wo, bo):
    npad = s.shape[0]
    blk = 1024
    return pl.pallas_call(
        _head_body,
        grid=(npad // blk,),
        in_specs=[
            pl.BlockSpec((blk, HID), lambda i: (i, 0)),
            pl.BlockSpec((1, HID), lambda i: (0, 0)),
            pl.BlockSpec((HID, 2), lambda i: (0, 0)),
            pl.BlockSpec((1, 2), lambda i: (0, 0)),
        ],
        out_specs=pl.BlockSpec((blk, 2), lambda i: (i, 0)),
        out_shape=jax.ShapeDtypeStruct((npad, 2), _f32),
    )(s, b1.reshape(1, HID), wo, bo.reshape(1, 2))


def _chunked_idx(idx, npad):
    n, l = idx.shape
    p = jnp.pad(idx, ((0, npad - n), (0, 0)))
    return p.reshape(npad // CH, CH, l).transpose(0, 2, 1)


def kernel(h, idx2, idx3, idx4, W1_2, b1_2, Wo_2, bo_2,
           W1_3, b1_3, Wo_3, bo_3, W1_4, b1_4, Wo_4, bo_4):
    c2 = W1_2[:D] + W1_2[D:]
    c3a = W1_3[:D] + W1_3[2 * D:]
    c3b = 2.0 * W1_3[D:2 * D]
    c4a = W1_4[:D] + W1_4[3 * D:]
    c4b = W1_4[D:2 * D] + W1_4[2 * D:3 * D]
    wc = jnp.concatenate([c2, c3a, c3b, c4a, c4b], axis=1)

    tables = _make_tables(h, wc)

    g2 = _chunked_idx(idx2, N2P)
    g3 = _chunked_idx(idx3, N3P)
    g4 = _chunked_idx(idx4, N4P)

    s2, s3, s4 = _sc_gather_sum(tables, g2, g3, g4)

    o2 = _head(s2, b1_2, Wo_2, bo_2)
    o3 = _head(s3, b1_3, Wo_3, bo_3)
    o4 = _head(s4, b1_4, Wo_4, bo_4)
    return jnp.concatenate([o2[:N2], o3[:N3], o4[:N4]], axis=0)
